# baseline (speedup 1.0000x reference)
import functools

import jax
import jax.numpy as jnp
from jax import lax
from jax.experimental import pallas as pl
from jax.experimental.pallas import tpu as pltpu
from jax.experimental.pallas import tpu_sc as plsc

N_CLASSES = 1000
BATCH = 16384

NC = 2
NS = 16
L = 16
NW = NC * NS
COLS_PER_W = BATCH // NW
C_BLK = 128
N_BLKS = COLS_PER_W // C_BLK
R_A = 496
R_B = N_CLASSES - R_A

_mesh = plsc.VectorSubcoreMesh(core_axis_name="c", subcore_axis_name="s")


@functools.partial(
    pl.kernel,
    out_type=jax.ShapeDtypeStruct((N_CLASSES, BATCH), jnp.float32),
    mesh=_mesh,
    scratch_types=[
        pltpu.VMEM((COLS_PER_W,), jnp.int32),
        pltpu.VMEM((R_A, C_BLK), jnp.float32),
        pltpu.VMEM((R_B, C_BLK), jnp.float32),
        pltpu.SemaphoreType.DMA,
        pltpu.SemaphoreType.DMA,
    ],
    compiler_params=pltpu.CompilerParams(needs_layout_passes=False),
)
def _one_hot_t_sc(idx_hbm, z_hbm, out_hbm, idx_v, buf_a, buf_b,
                  sem_a, sem_b):
    wid = lax.axis_index("s") * NC + lax.axis_index("c")
    col0 = wid * COLS_PER_W

    za = pltpu.async_copy(z_hbm.at[pl.ds(0, R_A)], buf_a, sem_a)
    zb = pltpu.async_copy(z_hbm.at[pl.ds(0, R_B)], buf_b, sem_b)
    pltpu.sync_copy(idx_hbm.at[pl.ds(col0, COLS_PER_W)], idx_v)
    za.wait()
    zb.wait()

    zeros16 = jnp.zeros((L,), jnp.float32)
    ones16 = jnp.ones((L,), jnp.float32)
    lane = lax.iota(jnp.int32, L)

    halves = (
        (buf_a, sem_a, 0, R_A),
        (buf_b, sem_b, R_A, R_B),
    )

    def _flip(half, blk, vals):
        buf, _, base, nr = halves[half]
        for g in range(C_BLK // L):
            idxv = idx_v[pl.ds(blk * C_BLK + g * L, L)]
            rel = idxv - base
            mask = (rel >= 0) & (rel < nr)
            plsc.store_scatter(buf, (rel, lane + (g * L)), vals, mask=mask)

    def _wait(half):
        buf, sem, base, nr = halves[half]
        pltpu.make_async_copy(
            buf, out_hbm.at[pl.ds(base, nr), pl.ds(0, C_BLK)], sem).wait()

    pend = [None, None]
    for c in range(N_BLKS):
        for h in (0, 1):
            buf, sem, base, nr = halves[h]
            if pend[h] is not None:
                _wait(h)
                _flip(h, pend[h], zeros16)
            _flip(h, c, ones16)
            pltpu.async_copy(
                buf,
                out_hbm.at[pl.ds(base, nr),
                           pl.ds(col0 + c * C_BLK, C_BLK)], sem)
            pend[h] = c
    _wait(0)
    _wait(1)


def kernel(inputs):
    idx = inputs.astype(jnp.int32)
    zblk = jnp.zeros((R_B, C_BLK), jnp.float32)
    out_t = _one_hot_t_sc(idx, zblk)
    return out_t.T


# speedup vs baseline: 1.1028x; 1.1028x over previous
import functools

import jax
import jax.numpy as jnp
from jax import lax
from jax.experimental import pallas as pl
from jax.experimental.pallas import tpu as pltpu
from jax.experimental.pallas import tpu_sc as plsc

N_CLASSES = 1000
BATCH = 16384

NC = 2
NS = 16
L = 16
NW = NC * NS
COLS_PER_W = BATCH // NW
C_BLK = 128
N_BLKS = COLS_PER_W // C_BLK

_mesh = plsc.VectorSubcoreMesh(core_axis_name="c", subcore_axis_name="s")


@functools.partial(
    pl.kernel,
    out_type=jax.ShapeDtypeStruct((N_CLASSES, BATCH), jnp.float32),
    mesh=_mesh,
    scratch_types=[
        pltpu.VMEM((COLS_PER_W,), jnp.int32),
        pltpu.VMEM((N_CLASSES, C_BLK), jnp.float32),
    ],
    compiler_params=pltpu.CompilerParams(needs_layout_passes=False),
)
def _one_hot_t_sc(idx_hbm, z_hbm, out_hbm, idx_v, buf):
    wid = lax.axis_index("s") * NC + lax.axis_index("c")
    col0 = wid * COLS_PER_W

    pltpu.sync_copy(idx_hbm.at[pl.ds(col0, COLS_PER_W)], idx_v)
    pltpu.sync_copy(z_hbm, buf)

    zeros16 = jnp.zeros((L,), jnp.float32)
    ones16 = jnp.ones((L,), jnp.float32)
    lane = lax.iota(jnp.int32, L)

    def _flip(blk, vals):
        for g in range(C_BLK // L):
            idxv = idx_v[pl.ds(blk * C_BLK + g * L, L)]
            plsc.store_scatter(buf, (idxv, lane + (g * L)), vals)

    for c in range(N_BLKS):
        _flip(c, ones16)
        pltpu.sync_copy(buf, out_hbm.at[:, pl.ds(col0 + c * C_BLK, C_BLK)])
        if c + 1 < N_BLKS:
            _flip(c, zeros16)


def kernel(inputs):
    idx = inputs.astype(jnp.int32)
    zblk = jnp.zeros((N_CLASSES, C_BLK), jnp.float32)
    out_t = _one_hot_t_sc(idx, zblk)
    return out_t.T
